# baseline (speedup 1.0000x reference)
import functools

import jax
import jax.numpy as jnp
from jax import lax
from jax.experimental import pallas as pl
from jax.experimental.pallas import tpu as pltpu
from jax.experimental.pallas import tpu_sc as plsc

N = 10000
E = 160000
F = 128
H = 8
NC = 2
NS = 16
NW = NC * NS
EPW = E // NW
NP = 10240
SUB = NP // NS


def _silu(x):
    return x * jax.nn.sigmoid(x)


def _node_body(s_ref, wq, bq, wk, bk, wg1, bg1, wg2, bg2, wv1, bv1, wv2, bv2,
               q_ref, k_ref, xv_ref):
    s = s_ref[...]
    dot = functools.partial(jnp.dot, preferred_element_type=jnp.float32)
    q_ref[...] = dot(s, wq[...]) + bq[...]
    k_ref[...] = dot(s, wk[...]) + bk[...]
    x = dot(_silu(dot(s, wg1[...]) + bg1[...]), wg2[...]) + bg2[...]
    v = dot(_silu(dot(s, wv1[...]) + bv1[...]), wv2[...]) + bv2[...]
    xv_ref[...] = x * v


def _ex_body(qd_ref, ks_ref, r_ref, wra, bra, hsum, exp16, ex_ref):
    dot = functools.partial(jnp.dot, preferred_element_type=jnp.float32)
    ra = _silu(dot(r_ref[...], wra[...]) + bra[...])
    logits = dot(qd_ref[...] * ks_ref[...] * ra, hsum[...])
    ex_ref[...] = dot(jnp.exp(logits), exp16[...])


def _densum_body(p_ref, den_ref):
    den_ref[...] = p_ref[0] + p_ref[1]


def _msg_body(xvs_ref, ts_ref, r_ref, ex_ref, dend_ref, d_ref, ne_ref,
              dir_ref, wre, bre, expand, sel8, msg_ref):
    dot = functools.partial(jnp.dot, preferred_element_type=jnp.float32)
    rf = dot(r_ref[...], wre[...]) + bre[...]
    d = d_ref[...]
    cut = 0.5 * (jnp.cos(d * (jnp.pi / 5.0)) + 1.0) * (d < 5.0).astype(d.dtype)
    scale = cut * jnp.sqrt(jnp.maximum(ne_ref[...], 1.0))
    attn8 = dot(ex_ref[...] / (dend_ref[...] + 1e-16), sel8[...]) * scale
    p = xvs_ref[...] * rf * dot(attn8, expand[...])
    p_d = p[:, F:2 * F]
    p_t = p[:, 2 * F:]
    parts = [p[:, :F]]
    for l in range(3):
        parts.append(p_d * dir_ref[:, l:l + 1] + p_t * ts_ref[:, l * F:(l + 1) * F])
    msg_ref[...] = jnp.concatenate(parts, axis=-1)


def _final_body(s_ref, t_ref, acc_ref, os_ref, ot_ref):
    os_ref[...] = s_ref[...] + acc_ref[0]
    ot_ref[...] = t_ref[...] + jnp.concatenate(
        [acc_ref[1], acc_ref[2], acc_ref[3]], axis=-1)


_MESH = plsc.VectorSubcoreMesh(core_axis_name="c", subcore_axis_name="s",
                               num_cores=NC, num_subcores=NS)


def _wid():
    return lax.axis_index("s") * NC + lax.axis_index("c")


def _qk_gather_body(q_hbm, k_hbm, dst_hbm, src_hbm, qd_hbm, ks_hbm,
                    idxd, idxs, qbuf, kbuf, sem1, sem2):
    base = _wid() * EPW
    C = 40

    @pl.loop(0, EPW // C)
    def _(i):
        off = pl.multiple_of(base + i * C, 8)
        pltpu.sync_copy(dst_hbm.at[pl.ds(off, C)], idxd)
        pltpu.sync_copy(src_hbm.at[pl.ds(off, C)], idxs)
        cp1 = pltpu.async_copy(q_hbm.at[idxd], qbuf, sem1)
        cp2 = pltpu.async_copy(k_hbm.at[idxs], kbuf, sem2)
        cp1.wait()
        cp2.wait()
        pltpu.sync_copy(qbuf, qd_hbm.at[pl.ds(off, C)])
        pltpu.sync_copy(kbuf, ks_hbm.at[pl.ds(off, C)])


def _den_scatter_body(ex_hbm, dst_hbm, z_hbm, den_hbm, idx, upd, den_sh):
    c = lax.axis_index("c")
    s = lax.axis_index("s")
    r0 = s * SUB
    pltpu.sync_copy(z_hbm.at[pl.ds(r0, SUB)], den_sh.at[pl.ds(r0, SUB)])
    plsc.subcore_barrier()
    base = _wid() * EPW
    C = 200

    @pl.loop(0, EPW // C)
    def _(i):
        off = pl.multiple_of(base + i * C, 8)
        pltpu.sync_copy(dst_hbm.at[pl.ds(off, C)], idx)
        pltpu.sync_copy(ex_hbm.at[pl.ds(off, C)], upd)
        pltpu.sync_copy(upd, den_sh.at[idx], add=True)

    plsc.subcore_barrier()
    pltpu.sync_copy(den_sh.at[pl.ds(r0, SUB)], den_hbm.at[c, pl.ds(r0, SUB)])


def _den_gather_body(den_hbm, dst_hbm, dend_hbm, idx, buf):
    base = _wid() * EPW
    C = 40

    @pl.loop(0, EPW // C)
    def _(i):
        off = pl.multiple_of(base + i * C, 8)
        pltpu.sync_copy(dst_hbm.at[pl.ds(off, C)], idx)
        pltpu.sync_copy(den_hbm.at[idx], buf)
        pltpu.sync_copy(buf, dend_hbm.at[pl.ds(off, C)])


def _xvt_gather_body(xv_hbm, t_hbm, src_hbm, xvs_hbm, ts_hbm, idx, buf):
    base = _wid() * EPW
    C = 40

    @pl.loop(0, EPW // C)
    def _(i):
        off = pl.multiple_of(base + i * C, 8)
        pltpu.sync_copy(src_hbm.at[pl.ds(off, C)], idx)
        pltpu.sync_copy(xv_hbm.at[idx], buf)
        pltpu.sync_copy(buf, xvs_hbm.at[pl.ds(off, C)])
        pltpu.sync_copy(t_hbm.at[idx], buf)
        pltpu.sync_copy(buf, ts_hbm.at[pl.ds(off, C)])


def _msg_scatter_body(msg_hbm, dst_hbm, z_hbm, acc_hbm, idx, upd, acc_sh):
    c = lax.axis_index("c")
    s = lax.axis_index("s")
    r0 = s * SUB
    epc = E // NS
    C = 200
    for j in range(2):
        cc = c * 2 + j
        pltpu.sync_copy(z_hbm.at[pl.ds(r0, SUB)], acc_sh.at[pl.ds(r0, SUB)])
        plsc.subcore_barrier()

        @pl.loop(0, epc // C)
        def _(i):
            off = pl.multiple_of(s * epc + i * C, 8)
            pltpu.sync_copy(dst_hbm.at[pl.ds(off, C)], idx)
            pltpu.sync_copy(msg_hbm.at[pl.ds(off, C), pl.ds(cc * F, F)], upd)
            pltpu.sync_copy(upd, acc_sh.at[idx], add=True)

        plsc.subcore_barrier()
        pltpu.sync_copy(acc_sh.at[pl.ds(r0, SUB)],
                        acc_hbm.at[cc, pl.ds(r0, SUB)])
        plsc.subcore_barrier()


def kernel(edge_index, s, t, dir_ij, r_ij, d_ij, num_edges_expanded,
           Wq, bq, Wk, bk, Wg1, bg1, Wg2, bg2, Wv1, bv1, Wv2, bv2,
           Wra, bra, Wre, bre):
    f32 = jnp.float32
    src = edge_index[0]
    dst = edge_index[1]
    t2 = t.reshape(N, 3 * F)
    d2 = d_ij.reshape(E, 1)
    ne2 = num_edges_expanded.reshape(E, 1)
    b2 = lambda b: b.reshape(1, -1)
    hsum = (jnp.arange(F)[:, None] // (F // H) == jnp.arange(H)[None, :]).astype(f32)
    exp16 = (jnp.arange(F)[None, :] // (F // H) == jnp.arange(H)[:, None]).astype(f32)
    sel8 = (jnp.arange(F)[:, None] == (F // H) * jnp.arange(H)[None, :]).astype(f32)
    expand = (jnp.arange(3 * F)[None, :] // (3 * F // H) == jnp.arange(H)[:, None]).astype(f32)
    zacc = jnp.zeros((NP, F), f32)

    full = lambda shape: pl.BlockSpec(shape, lambda i: tuple(0 for _ in shape))

    q, k, xv = pl.pallas_call(
        _node_body,
        out_shape=(jax.ShapeDtypeStruct((N, F), f32),
                   jax.ShapeDtypeStruct((N, F), f32),
                   jax.ShapeDtypeStruct((N, 3 * F), f32)),
        grid=(25,),
        in_specs=[pl.BlockSpec((400, F), lambda i: (i, 0))] +
                 [full(shp) for shp in
                  ((128, 128), (1, 128), (128, 128), (1, 128),
                   (128, 128), (1, 128), (128, 384), (1, 384),
                   (128, 128), (1, 128), (128, 384), (1, 384))],
        out_specs=(pl.BlockSpec((400, F), lambda i: (i, 0)),
                   pl.BlockSpec((400, F), lambda i: (i, 0)),
                   pl.BlockSpec((400, 3 * F), lambda i: (i, 0))),
    )(s, Wq, b2(bq), Wk, b2(bk), Wg1, b2(bg1), Wg2, b2(bg2),
      Wv1, b2(bv1), Wv2, b2(bv2))

    qk_gather = pl.kernel(
        _qk_gather_body,
        out_type=(jax.ShapeDtypeStruct((E, F), f32),
                  jax.ShapeDtypeStruct((E, F), f32)),
        mesh=_MESH,
        scratch_types=[pltpu.VMEM((40,), jnp.int32),
                       pltpu.VMEM((40,), jnp.int32),
                       pltpu.VMEM((40, F), f32),
                       pltpu.VMEM((40, F), f32),
                       pltpu.SemaphoreType.DMA,
                       pltpu.SemaphoreType.DMA],
    )
    qd, ks = qk_gather(q, k, dst, src)

    ex = pl.pallas_call(
        _ex_body,
        out_shape=jax.ShapeDtypeStruct((E, F), f32),
        grid=(100,),
        in_specs=[pl.BlockSpec((1600, F), lambda i: (i, 0)),
                  pl.BlockSpec((1600, F), lambda i: (i, 0)),
                  pl.BlockSpec((1600, F), lambda i: (i, 0)),
                  full((128, 128)), full((1, 128)), full((128, 8)),
                  full((8, 128))],
        out_specs=pl.BlockSpec((1600, F), lambda i: (i, 0)),
    )(qd, ks, r_ij, Wra, b2(bra), hsum, exp16)

    den_scatter = pl.kernel(
        _den_scatter_body,
        out_type=jax.ShapeDtypeStruct((NC, NP, F), f32),
        mesh=_MESH,
        scratch_types=[pltpu.VMEM((200,), jnp.int32),
                       pltpu.VMEM((200, F), f32),
                       pltpu.VMEM_SHARED((NP, F), f32)],
    )
    den_parts = den_scatter(ex, dst, zacc)

    den = pl.pallas_call(
        _densum_body,
        out_shape=jax.ShapeDtypeStruct((NP, F), f32),
        grid=(20,),
        in_specs=[pl.BlockSpec((NC, 512, F), lambda i: (0, i, 0))],
        out_specs=pl.BlockSpec((512, F), lambda i: (i, 0)),
    )(den_parts)

    den_gather = pl.kernel(
        _den_gather_body,
        out_type=jax.ShapeDtypeStruct((E, F), f32),
        mesh=_MESH,
        scratch_types=[pltpu.VMEM((40,), jnp.int32),
                       pltpu.VMEM((40, F), f32)],
    )
    dend = den_gather(den, dst)

    xvt_gather = pl.kernel(
        _xvt_gather_body,
        out_type=(jax.ShapeDtypeStruct((E, 3 * F), f32),
                  jax.ShapeDtypeStruct((E, 3 * F), f32)),
        mesh=_MESH,
        scratch_types=[pltpu.VMEM((40,), jnp.int32),
                       pltpu.VMEM((40, 3 * F), f32)],
    )
    xvs, ts = xvt_gather(xv, t2, src)

    msg = pl.pallas_call(
        _msg_body,
        out_shape=jax.ShapeDtypeStruct((E, 4 * F), f32),
        grid=(200,),
        in_specs=[pl.BlockSpec((800, 3 * F), lambda i: (i, 0)),
                  pl.BlockSpec((800, 3 * F), lambda i: (i, 0)),
                  pl.BlockSpec((800, F), lambda i: (i, 0)),
                  pl.BlockSpec((800, F), lambda i: (i, 0)),
                  pl.BlockSpec((800, F), lambda i: (i, 0)),
                  pl.BlockSpec((800, 1), lambda i: (i, 0)),
                  pl.BlockSpec((800, 1), lambda i: (i, 0)),
                  pl.BlockSpec((800, 3), lambda i: (i, 0)),
                  full((128, 384)), full((1, 384)), full((8, 384)),
                  full((128, 8))],
        out_specs=pl.BlockSpec((800, 4 * F), lambda i: (i, 0)),
    )(xvs, ts, r_ij, ex, dend, d2, ne2, dir_ij, Wre, b2(bre), expand, sel8)

    msg_scatter = pl.kernel(
        _msg_scatter_body,
        out_type=jax.ShapeDtypeStruct((4, NP, F), f32),
        mesh=_MESH,
        scratch_types=[pltpu.VMEM((200,), jnp.int32),
                       pltpu.VMEM((200, F), f32),
                       pltpu.VMEM_SHARED((NP, F), f32)],
    )
    acc = msg_scatter(msg, dst, zacc)

    out_s, out_t2 = pl.pallas_call(
        _final_body,
        out_shape=(jax.ShapeDtypeStruct((N, F), f32),
                   jax.ShapeDtypeStruct((N, 3 * F), f32)),
        grid=(25,),
        in_specs=[pl.BlockSpec((400, F), lambda i: (i, 0)),
                  pl.BlockSpec((400, 3 * F), lambda i: (i, 0)),
                  pl.BlockSpec((4, 400, F), lambda i: (0, i, 0))],
        out_specs=(pl.BlockSpec((400, F), lambda i: (i, 0)),
                   pl.BlockSpec((400, 3 * F), lambda i: (i, 0))),
    )(s, t2, acc)

    return (out_s, out_t2.reshape(N, 3, F))
